# baseline (speedup 1.0000x reference)
import functools

import jax
import jax.numpy as jnp
from jax import lax
from jax.experimental import pallas as pl
from jax.experimental.pallas import tpu as pltpu
from jax.experimental.pallas import tpu_sc as plsc

_NUM_EDGES = 800000
_EMB_DIM = 64
_NW = 32
_BUF_COLS = 6400
_N_COL_CHUNKS = _NUM_EDGES // _BUF_COLS
_TILES_PER_RB = 4
_PER_TILE = 32
_FIRE = 32


def _tc_lookup(mid_ref, tableT_ref, out_ref, stage_v, sem):
    c = mid_ref[0] % 128
    lane = jax.lax.broadcasted_iota(jnp.int32, (1, 128), 1)
    masked = jnp.where(lane == c, tableT_ref[...], 0.0)
    col = jnp.sum(masked, axis=1, keepdims=True)
    stage_v[...] = jnp.broadcast_to(col, stage_v.shape)
    pltpu.make_async_copy(stage_v, out_ref, sem).start()
    pltpu.make_async_copy(stage_v, out_ref, sem).wait()


def _sc_body(stage_hbm, out_hbm, stage_v, buf_v, sem_o):
    wid = lax.axis_index("s") * 2 + lax.axis_index("c")
    rb = pl.multiple_of((wid % 8) * 8, 8)
    q = wid // 8

    pltpu.sync_copy(stage_hbm, stage_v)
    regs = tuple(stage_v[rb + j, pl.ds(0, 16)] for j in range(8))

    def fill(t, carry):
        for j in range(8):
            buf_v[j, pl.ds(t * 16, 16)] = carry[j]
        return carry

    lax.fori_loop(0, _BUF_COLS // 16, fill, regs)

    first = q * 31
    for g in range(0, _PER_TILE, _FIRE):
        copies = []
        for t in range(g, min(g + _FIRE, _PER_TILE)):
            cc = pl.multiple_of(
                jnp.minimum(first + t, _N_COL_CHUNKS - 1) * _BUF_COLS, _BUF_COLS
            )
            copies.append(
                pltpu.make_async_copy(
                    buf_v,
                    out_hbm.at[pl.ds(rb, 8), pl.ds(cc, _BUF_COLS)],
                    sem_o,
                )
            )
        for cpy in copies:
            cpy.start()
        for cpy in copies:
            cpy.wait()


def kernel(material_id, num_edges, table):
    del num_edges
    tableT = jnp.swapaxes(table, 0, 1)

    stage = pl.pallas_call(
        _tc_lookup,
        grid_spec=pltpu.PrefetchScalarGridSpec(
            num_scalar_prefetch=1,
            grid=(1,),
            in_specs=[
                pl.BlockSpec((_EMB_DIM, 128), lambda i, mid: (0, mid[0] // 128)),
            ],
            out_specs=pl.BlockSpec(memory_space=pl.ANY),
            scratch_shapes=[
                pltpu.VMEM((_EMB_DIM, 16), jnp.float32),
                pltpu.SemaphoreType.DMA,
            ],
        ),
        out_shape=jax.ShapeDtypeStruct((_EMB_DIM, 16), jnp.float32),
    )(material_id, tableT)

    mesh = plsc.VectorSubcoreMesh(core_axis_name="c", subcore_axis_name="s")
    kern = functools.partial(
        pl.kernel,
        mesh=mesh,
        out_type=jax.ShapeDtypeStruct((_EMB_DIM, _NUM_EDGES), jnp.float32),
        scratch_types=[
            pltpu.VMEM((_EMB_DIM, 16), jnp.float32),
            pltpu.VMEM((8, _BUF_COLS), jnp.float32),
            pltpu.SemaphoreType.DMA,
        ],
    )(_sc_body)
    wide = kern(stage)
    return jnp.swapaxes(wide, 0, 1)


# speedup vs baseline: 1.0147x; 1.0147x over previous
import functools

import jax
import jax.numpy as jnp
from jax import lax
from jax.experimental import pallas as pl
from jax.experimental.pallas import tpu as pltpu
from jax.experimental.pallas import tpu_sc as plsc

_NUM_EDGES = 800000
_EMB_DIM = 64
_NW = 32
_BUF_COLS = 3200
_N_COL_CHUNKS = _NUM_EDGES // _BUF_COLS
_TILES_PER_RB = 4
_PER_TILE = 63
_FIRE = 63


def _tc_lookup(mid_ref, tableT_ref, out_ref, stage_v, sem):
    c = mid_ref[0] % 128
    lane = jax.lax.broadcasted_iota(jnp.int32, (1, 128), 1)
    masked = jnp.where(lane == c, tableT_ref[...], 0.0)
    col = jnp.sum(masked, axis=1, keepdims=True)
    stage_v[...] = jnp.broadcast_to(col, stage_v.shape)
    pltpu.make_async_copy(stage_v, out_ref, sem).start()
    pltpu.make_async_copy(stage_v, out_ref, sem).wait()


def _sc_body(stage_hbm, out_hbm, stage_v, buf_v, sem_o):
    wid = lax.axis_index("s") * 2 + lax.axis_index("c")
    rb = pl.multiple_of((wid % 8) * 8, 8)
    q = wid // 8

    pltpu.sync_copy(stage_hbm, stage_v)
    regs = tuple(stage_v[rb + j, pl.ds(0, 16)] for j in range(8))

    def fill(t, carry):
        for j in range(8):
            buf_v[j, pl.ds(t * 16, 16)] = carry[j]
        return carry

    lax.fori_loop(0, _BUF_COLS // 16, fill, regs)

    first = q * 62
    for g in range(0, _PER_TILE, _FIRE):
        copies = []
        for t in range(g, min(g + _FIRE, _PER_TILE)):
            cc = pl.multiple_of(
                jnp.minimum(first + t, _N_COL_CHUNKS - 1) * _BUF_COLS, _BUF_COLS
            )
            copies.append(
                pltpu.make_async_copy(
                    buf_v,
                    out_hbm.at[pl.ds(rb, 8), pl.ds(cc, _BUF_COLS)],
                    sem_o,
                )
            )
        for cpy in copies:
            cpy.start()
        for cpy in copies:
            cpy.wait()


def kernel(material_id, num_edges, table):
    del num_edges
    tableT = jnp.swapaxes(table, 0, 1)

    stage = pl.pallas_call(
        _tc_lookup,
        grid_spec=pltpu.PrefetchScalarGridSpec(
            num_scalar_prefetch=1,
            grid=(1,),
            in_specs=[
                pl.BlockSpec((_EMB_DIM, 128), lambda i, mid: (0, mid[0] // 128)),
            ],
            out_specs=pl.BlockSpec(memory_space=pl.ANY),
            scratch_shapes=[
                pltpu.VMEM((_EMB_DIM, 16), jnp.float32),
                pltpu.SemaphoreType.DMA,
            ],
        ),
        out_shape=jax.ShapeDtypeStruct((_EMB_DIM, 16), jnp.float32),
    )(material_id, tableT)

    mesh = plsc.VectorSubcoreMesh(core_axis_name="c", subcore_axis_name="s")
    kern = functools.partial(
        pl.kernel,
        mesh=mesh,
        out_type=jax.ShapeDtypeStruct((_EMB_DIM, _NUM_EDGES), jnp.float32),
        scratch_types=[
            pltpu.VMEM((_EMB_DIM, 16), jnp.float32),
            pltpu.VMEM((8, _BUF_COLS), jnp.float32),
            pltpu.SemaphoreType.DMA,
        ],
    )(_sc_body)
    wide = kern(stage)
    return jnp.swapaxes(wide, 0, 1)
